# Initial kernel scaffold; baseline (speedup 1.0000x reference)
import jax
import jax.numpy as jnp
from jax.experimental import pallas as pl


def kernel(node_indices, positions, adjacency, mask, species_energy, pair_weight):
    raise NotImplementedError("write your pallas kernel here")


# speedup vs baseline: 2.6844x; 2.6844x over previous
import jax
import jax.numpy as jnp
from jax.experimental import pallas as pl
from jax.experimental.pallas import tpu as pltpu

_TM = 512


def _energy_body(idx_ref, pr_ref, pc_ref, mr_ref, mc_ref, se_ref, adj_ref,
                 pair_ref, atom_ref):
    i = pl.program_id(1)

    @pl.when(i == 0)
    def _init():
        pair_ref[...] = jnp.zeros_like(pair_ref)
        atom_ref[...] = jnp.zeros_like(atom_ref)

    pr = pr_ref[0]
    pc = pc_ref[0]
    mr = mr_ref[0]
    mc = mc_ref[0]

    r2r = jnp.sum(pr * pr, axis=1, keepdims=True)
    r2c = jnp.sum(pc * pc, axis=0, keepdims=True)
    g = jax.lax.dot_general(pr, pc, (((1,), (0,)), ((), ())),
                            preferred_element_type=jnp.float32)
    d2 = (r2r + r2c) - 2.0 * g
    dist = jnp.sqrt(jnp.maximum(d2, 0.0))
    t = adj_ref[0].astype(jnp.float32) * dist * mc
    rs = jnp.sum(t, axis=1, keepdims=True)
    pair_ref[...] = pair_ref[...] + jnp.sum(rs * mr)

    onehot = (jax.lax.broadcasted_iota(jnp.int32, (idx_ref.shape[1], 128), 1)
              == idx_ref[0]).astype(jnp.float32)
    ae = jnp.dot(onehot, se_ref[...], preferred_element_type=jnp.float32)
    atom_ref[...] = atom_ref[...] + jnp.sum(ae * mr)


def kernel(node_indices, positions, adjacency, mask, species_energy,
           pair_weight):
    B, N = node_indices.shape
    S = species_energy.shape[0]
    TM = _TM

    maskf = mask.astype(jnp.float32)
    mask_row = maskf.reshape(B, N, 1)
    mask_col = maskf.reshape(B, 1, N)
    idx2 = node_indices.astype(jnp.int32).reshape(B, N, 1)
    pos_c = positions.transpose(0, 2, 1)
    se = jnp.zeros((128, 1), jnp.float32).at[:S, 0].set(species_energy)

    grid = (B, N // TM)
    pair, atom = pl.pallas_call(
        _energy_body,
        grid=grid,
        in_specs=[
            pl.BlockSpec((1, TM, 1), lambda b, i: (b, i, 0)),
            pl.BlockSpec((1, TM, 3), lambda b, i: (b, i, 0)),
            pl.BlockSpec((1, 3, N), lambda b, i: (b, 0, 0)),
            pl.BlockSpec((1, TM, 1), lambda b, i: (b, i, 0)),
            pl.BlockSpec((1, 1, N), lambda b, i: (b, 0, 0)),
            pl.BlockSpec((128, 1), lambda b, i: (0, 0)),
            pl.BlockSpec((1, TM, N), lambda b, i: (b, i, 0)),
        ],
        out_specs=[
            pl.BlockSpec((1, 8, 128), lambda b, i: (b, 0, 0)),
            pl.BlockSpec((1, 8, 128), lambda b, i: (b, 0, 0)),
        ],
        out_shape=[
            jax.ShapeDtypeStruct((B, 8, 128), jnp.float32),
            jax.ShapeDtypeStruct((B, 8, 128), jnp.float32),
        ],
        compiler_params=pltpu.CompilerParams(
            dimension_semantics=("parallel", "arbitrary")),
    )(idx2, positions, pos_c, mask_row, mask_col, se, adjacency)

    return atom[:, 0, 0] + pair_weight * pair[:, 0, 0]
